# Initial kernel scaffold; baseline (speedup 1.0000x reference)
import jax
import jax.numpy as jnp
from jax.experimental import pallas as pl


def kernel(x, edge_index, W1, b1, W2, b2, Wout, bout):
    raise NotImplementedError("write your pallas kernel here")


# speedup vs baseline: 8.2110x; 8.2110x over previous
import functools

import jax
import jax.numpy as jnp
from jax import lax
from jax.experimental import pallas as pl
from jax.experimental.pallas import tpu as pltpu
from jax.experimental.pallas import tpu_sc as plsc

N = 10000
E = 320000
D = 128

NC = 2
NS = 16
NW = NC * NS

CHUNK = 128
CPT = 80
EPT = CPT * CHUNK
E_PAD = NW * EPT

AGG_PAD = 10240
AGG_TILE = AGG_PAD // NS
DEG_PAD = 10240
DEG_TILE = DEG_PAD // NS

R = 1000
G = N // R


def _mesh():
    return plsc.VectorSubcoreMesh(core_axis_name="c", subcore_axis_name="s")


@functools.partial(
    pl.kernel,
    out_type=jax.ShapeDtypeStruct((NC, DEG_PAD), jnp.float32),
    mesh=_mesh(),
    scratch_types=[
        pltpu.VMEM((CPT, CHUNK), jnp.int32),
        pltpu.VMEM((CHUNK,), jnp.float32),
        pltpu.VMEM_SHARED((DEG_PAD,), jnp.float32),
    ],
)
def _deg_sc(dst_hbm, z_hbm, ones_hbm, out_hbm, dst_v, ones_v, acc_sh):
    cid = lax.axis_index("c")
    sid = lax.axis_index("s")
    wid = cid * NS + sid
    pltpu.sync_copy(dst_hbm.at[pl.ds(wid * CPT, CPT)], dst_v)
    pltpu.sync_copy(ones_hbm, ones_v)
    pltpu.sync_copy(z_hbm.at[pl.ds(sid * DEG_TILE, DEG_TILE)],
                    acc_sh.at[pl.ds(sid * DEG_TILE, DEG_TILE)])
    plsc.subcore_barrier()

    def body(j, carry):
        pltpu.sync_copy(ones_v, acc_sh.at[dst_v.at[j]], add=True)
        return carry

    lax.fori_loop(0, CPT, body, 0)
    plsc.subcore_barrier()
    pltpu.sync_copy(acc_sh.at[pl.ds(sid * DEG_TILE, DEG_TILE)],
                    out_hbm.at[cid].at[pl.ds(sid * DEG_TILE, DEG_TILE)])


@functools.partial(
    pl.kernel,
    out_type=jax.ShapeDtypeStruct((NC, AGG_PAD, D), jnp.float32),
    mesh=_mesh(),
    scratch_types=[
        pltpu.VMEM((CPT, CHUNK), jnp.int32),
        pltpu.VMEM((CPT, CHUNK), jnp.int32),
        pltpu.VMEM((CHUNK, D), jnp.float32),
        pltpu.VMEM_SHARED((AGG_PAD, D), jnp.float32),
        pltpu.SemaphoreType.DMA,
    ],
)
def _agg_sc(g_hbm, src_hbm, dst_hbm, z_hbm, out_hbm,
            src_v, dst_v, rows_v, acc_sh, sem):
    cid = lax.axis_index("c")
    sid = lax.axis_index("s")
    wid = cid * NS + sid
    pltpu.sync_copy(src_hbm.at[pl.ds(wid * CPT, CPT)], src_v)
    pltpu.sync_copy(dst_hbm.at[pl.ds(wid * CPT, CPT)], dst_v)
    pltpu.sync_copy(z_hbm.at[pl.ds(sid * AGG_TILE, AGG_TILE)],
                    acc_sh.at[pl.ds(sid * AGG_TILE, AGG_TILE)])
    plsc.subcore_barrier()

    def body(j, carry):
        pltpu.async_copy(g_hbm.at[src_v.at[j]], rows_v, sem).wait()
        pltpu.sync_copy(rows_v, acc_sh.at[dst_v.at[j]], add=True)
        return carry

    lax.fori_loop(0, CPT, body, 0)
    plsc.subcore_barrier()
    pltpu.sync_copy(acc_sh.at[pl.ds(sid * AGG_TILE, AGG_TILE)],
                    out_hbm.at[cid].at[pl.ds(sid * AGG_TILE, AGG_TILE)])


def _dense1(x, W1, d0, d1):
    def body(x_ref, w_ref, d0_ref, d1_ref, h_ref, g_ref, dv_ref):
        dv = lax.rsqrt(d0_ref[...] + d1_ref[...] + 1.0)
        h = jnp.dot(x_ref[...], w_ref[...], preferred_element_type=jnp.float32)
        h_ref[...] = h
        g_ref[...] = h * dv
        dv_ref[...] = dv

    return pl.pallas_call(
        body,
        grid=(G,),
        in_specs=[
            pl.BlockSpec((R, D), lambda i: (i, 0)),
            pl.BlockSpec((D, D), lambda i: (0, 0)),
            pl.BlockSpec((R, 1), lambda i: (i, 0)),
            pl.BlockSpec((R, 1), lambda i: (i, 0)),
        ],
        out_specs=[
            pl.BlockSpec((R, D), lambda i: (i, 0)),
            pl.BlockSpec((R, D), lambda i: (i, 0)),
            pl.BlockSpec((R, 1), lambda i: (i, 0)),
        ],
        out_shape=[
            jax.ShapeDtypeStruct((N, D), jnp.float32),
            jax.ShapeDtypeStruct((N, D), jnp.float32),
            jax.ShapeDtypeStruct((N, 1), jnp.float32),
        ],
    )(x, W1, d0, d1)


def _dense_mid(p0, p1, h1, dv, b1, W2):
    def body(p0_ref, p1_ref, h_ref, dv_ref, b_ref, w_ref, h2_ref, g2_ref):
        dv = dv_ref[...]
        z = dv * (p0_ref[...] + p1_ref[...]) + dv * dv * h_ref[...] + b_ref[...]
        z = jnp.maximum(z, 0.0)
        h2 = jnp.dot(z, w_ref[...], preferred_element_type=jnp.float32)
        h2_ref[...] = h2
        g2_ref[...] = h2 * dv

    return pl.pallas_call(
        body,
        grid=(G,),
        in_specs=[
            pl.BlockSpec((R, D), lambda i: (i, 0)),
            pl.BlockSpec((R, D), lambda i: (i, 0)),
            pl.BlockSpec((R, D), lambda i: (i, 0)),
            pl.BlockSpec((R, 1), lambda i: (i, 0)),
            pl.BlockSpec((1, D), lambda i: (0, 0)),
            pl.BlockSpec((D, D), lambda i: (0, 0)),
        ],
        out_specs=[
            pl.BlockSpec((R, D), lambda i: (i, 0)),
            pl.BlockSpec((R, D), lambda i: (i, 0)),
        ],
        out_shape=[
            jax.ShapeDtypeStruct((N, D), jnp.float32),
            jax.ShapeDtypeStruct((N, D), jnp.float32),
        ],
    )(p0, p1, h1, dv, b1, W2)


def _dense_out(q0, q1, h2, dv, b2, Wout, bout):
    def body(q0_ref, q1_ref, h_ref, dv_ref, b_ref, w_ref, bo_ref, o_ref):
        dv = dv_ref[...]
        z = dv * (q0_ref[...] + q1_ref[...]) + dv * dv * h_ref[...] + b_ref[...]
        z = jnp.maximum(z, 0.0)
        o_ref[...] = jnp.dot(z, w_ref[...],
                             preferred_element_type=jnp.float32) + bo_ref[...]

    return pl.pallas_call(
        body,
        grid=(G,),
        in_specs=[
            pl.BlockSpec((R, D), lambda i: (i, 0)),
            pl.BlockSpec((R, D), lambda i: (i, 0)),
            pl.BlockSpec((R, D), lambda i: (i, 0)),
            pl.BlockSpec((R, 1), lambda i: (i, 0)),
            pl.BlockSpec((1, D), lambda i: (0, 0)),
            pl.BlockSpec((D, 1), lambda i: (0, 0)),
            pl.BlockSpec((1, 1), lambda i: (0, 0)),
        ],
        out_specs=pl.BlockSpec((R, 1), lambda i: (i, 0)),
        out_shape=jax.ShapeDtypeStruct((N, 1), jnp.float32),
    )(q0, q1, h2, dv, b2, Wout, bout)


def kernel(x, edge_index, W1, b1, W2, b2, Wout, bout):
    src = edge_index[0].astype(jnp.int32)
    dst = edge_index[1].astype(jnp.int32)
    pad = E_PAD - E
    src2d = jnp.concatenate([src, jnp.zeros((pad,), jnp.int32)]).reshape(NW * CPT, CHUNK)
    dst2d = jnp.concatenate([dst, jnp.full((pad,), N, jnp.int32)]).reshape(NW * CPT, CHUNK)
    zeros2d = jnp.zeros((AGG_PAD, D), jnp.float32)
    zeros1d = jnp.zeros((DEG_PAD,), jnp.float32)
    ones128 = jnp.ones((CHUNK,), jnp.float32)

    degp = _deg_sc(dst2d, zeros1d, ones128)
    d0 = degp[0, :N].reshape(N, 1)
    d1 = degp[1, :N].reshape(N, 1)

    h1, g1, dv = _dense1(x, W1, d0, d1)
    agg1 = _agg_sc(g1, src2d, dst2d, zeros2d)
    h2, g2 = _dense_mid(agg1[0, :N], agg1[1, :N], h1, dv,
                        b1.reshape(1, D), W2)
    agg2 = _agg_sc(g2, src2d, dst2d, zeros2d)
    return _dense_out(agg2[0, :N], agg2[1, :N], h2, dv,
                      b2.reshape(1, D), Wout, bout.reshape(1, 1))


# speedup vs baseline: 21.1860x; 2.5802x over previous
import functools

import jax
import jax.numpy as jnp
from jax import lax
from jax.experimental import pallas as pl
from jax.experimental.pallas import tpu as pltpu
from jax.experimental.pallas import tpu_sc as plsc

N = 10000
E = 320000
D = 128

NC = 2
NS = 16
NW = NC * NS

CHUNK = 128
CPT = 80
EPT = CPT * CHUNK
E_PAD = NW * EPT

AGG_PAD = 10240
AGG_TILE = AGG_PAD // NS
DEG_PAD = 10240
DEG_TILE = DEG_PAD // NS

R = 1000
G = N // R


def _mesh():
    return plsc.VectorSubcoreMesh(core_axis_name="c", subcore_axis_name="s")


@functools.partial(
    pl.kernel,
    out_type=jax.ShapeDtypeStruct((NC, DEG_PAD), jnp.float32),
    mesh=_mesh(),
    scratch_types=[
        pltpu.VMEM((CPT, CHUNK), jnp.int32),
        pltpu.VMEM((CHUNK,), jnp.float32),
        pltpu.VMEM_SHARED((DEG_PAD,), jnp.float32),
    ],
)
def _deg_sc(dst_hbm, z_hbm, ones_hbm, out_hbm, dst_v, ones_v, acc_sh):
    cid = lax.axis_index("c")
    sid = lax.axis_index("s")
    wid = cid * NS + sid
    pltpu.sync_copy(dst_hbm.at[pl.ds(wid * CPT, CPT)], dst_v)
    pltpu.sync_copy(ones_hbm, ones_v)
    pltpu.sync_copy(z_hbm.at[pl.ds(sid * DEG_TILE, DEG_TILE)],
                    acc_sh.at[pl.ds(sid * DEG_TILE, DEG_TILE)])
    plsc.subcore_barrier()

    def body(j, carry):
        pltpu.sync_copy(ones_v, acc_sh.at[dst_v.at[j]], add=True)
        return carry

    lax.fori_loop(0, CPT, body, 0)
    plsc.subcore_barrier()
    pltpu.sync_copy(acc_sh.at[pl.ds(sid * DEG_TILE, DEG_TILE)],
                    out_hbm.at[cid].at[pl.ds(sid * DEG_TILE, DEG_TILE)])


@functools.partial(
    pl.kernel,
    out_type=jax.ShapeDtypeStruct((NC, AGG_PAD, D), jnp.float32),
    mesh=_mesh(),
    scratch_types=[
        pltpu.VMEM((CPT, CHUNK), jnp.int32),
        pltpu.VMEM((CPT, CHUNK), jnp.int32),
        pltpu.VMEM((CHUNK, D), jnp.float32),
        pltpu.VMEM_SHARED((AGG_PAD, D), jnp.float32),
        pltpu.SemaphoreType.DMA,
    ],
)
def _agg_sc(g_hbm, src_hbm, dst_hbm, z_hbm, out_hbm,
            src_v, dst_v, rows_v, acc_sh, sem):
    cid = lax.axis_index("c")
    sid = lax.axis_index("s")
    wid = cid * NS + sid
    pltpu.sync_copy(src_hbm.at[pl.ds(wid * CPT, CPT)], src_v)
    pltpu.sync_copy(dst_hbm.at[pl.ds(wid * CPT, CPT)], dst_v)
    pltpu.sync_copy(z_hbm.at[pl.ds(sid * AGG_TILE, AGG_TILE)],
                    acc_sh.at[pl.ds(sid * AGG_TILE, AGG_TILE)])
    plsc.subcore_barrier()

    def body(j, carry):
        pltpu.async_copy(g_hbm.at[src_v.at[j]], rows_v, sem).wait()
        pltpu.sync_copy(rows_v, acc_sh.at[dst_v.at[j]], add=True)
        return carry

    lax.fori_loop(0, CPT, body, 0)
    plsc.subcore_barrier()
    pltpu.sync_copy(acc_sh.at[pl.ds(sid * AGG_TILE, AGG_TILE)],
                    out_hbm.at[cid].at[pl.ds(sid * AGG_TILE, AGG_TILE)])


def _dense1(x, W1, d0, d1):
    def body(x_ref, w_ref, d0_ref, d1_ref, h_ref, g_ref, dv_ref):
        dv = lax.rsqrt(d0_ref[...] + d1_ref[...] + 1.0)
        h = jnp.dot(x_ref[...], w_ref[...], preferred_element_type=jnp.float32)
        h_ref[...] = h
        g_ref[...] = h * dv
        dv_ref[...] = dv

    return pl.pallas_call(
        body,
        grid=(G,),
        in_specs=[
            pl.BlockSpec((R, D), lambda i: (i, 0)),
            pl.BlockSpec((D, D), lambda i: (0, 0)),
            pl.BlockSpec((R, 1), lambda i: (i, 0)),
            pl.BlockSpec((R, 1), lambda i: (i, 0)),
        ],
        out_specs=[
            pl.BlockSpec((R, D), lambda i: (i, 0)),
            pl.BlockSpec((R, D), lambda i: (i, 0)),
            pl.BlockSpec((R, 1), lambda i: (i, 0)),
        ],
        out_shape=[
            jax.ShapeDtypeStruct((N, D), jnp.float32),
            jax.ShapeDtypeStruct((N, D), jnp.float32),
            jax.ShapeDtypeStruct((N, 1), jnp.float32),
        ],
    )(x, W1, d0, d1)


def _dense_mid(p0, p1, h1, dv, b1, W2):
    def body(p0_ref, p1_ref, h_ref, dv_ref, b_ref, w_ref, h2_ref, g2_ref):
        dv = dv_ref[...]
        z = dv * (p0_ref[...] + p1_ref[...]) + dv * dv * h_ref[...] + b_ref[...]
        z = jnp.maximum(z, 0.0)
        h2 = jnp.dot(z, w_ref[...], preferred_element_type=jnp.float32)
        h2_ref[...] = h2
        g2_ref[...] = h2 * dv

    return pl.pallas_call(
        body,
        grid=(G,),
        in_specs=[
            pl.BlockSpec((R, D), lambda i: (i, 0)),
            pl.BlockSpec((R, D), lambda i: (i, 0)),
            pl.BlockSpec((R, D), lambda i: (i, 0)),
            pl.BlockSpec((R, 1), lambda i: (i, 0)),
            pl.BlockSpec((1, D), lambda i: (0, 0)),
            pl.BlockSpec((D, D), lambda i: (0, 0)),
        ],
        out_specs=[
            pl.BlockSpec((R, D), lambda i: (i, 0)),
            pl.BlockSpec((R, D), lambda i: (i, 0)),
        ],
        out_shape=[
            jax.ShapeDtypeStruct((N, D), jnp.float32),
            jax.ShapeDtypeStruct((N, D), jnp.float32),
        ],
    )(p0, p1, h1, dv, b1, W2)


def _dense_out(q0, q1, h2, dv, b2, Wout, bout):
    def body(q0_ref, q1_ref, h_ref, dv_ref, b_ref, w_ref, bo_ref, o_ref):
        dv = dv_ref[...]
        z = dv * (q0_ref[...] + q1_ref[...]) + dv * dv * h_ref[...] + b_ref[...]
        z = jnp.maximum(z, 0.0)
        o_ref[...] = jnp.dot(z, w_ref[...],
                             preferred_element_type=jnp.float32) + bo_ref[...]

    return pl.pallas_call(
        body,
        grid=(G,),
        in_specs=[
            pl.BlockSpec((R, D), lambda i: (i, 0)),
            pl.BlockSpec((R, D), lambda i: (i, 0)),
            pl.BlockSpec((R, D), lambda i: (i, 0)),
            pl.BlockSpec((R, 1), lambda i: (i, 0)),
            pl.BlockSpec((1, D), lambda i: (0, 0)),
            pl.BlockSpec((D, 1), lambda i: (0, 0)),
            pl.BlockSpec((1, 1), lambda i: (0, 0)),
        ],
        out_specs=pl.BlockSpec((R, 1), lambda i: (i, 0)),
        out_shape=jax.ShapeDtypeStruct((N, 1), jnp.float32),
    )(q0, q1, h2, dv, b2, Wout, bout)


def kernel(x, edge_index, W1, b1, W2, b2, Wout, bout):
    src = edge_index[0].astype(jnp.int32)
    dst = edge_index[1].astype(jnp.int32)
    pad = E_PAD - E
    iota = lax.iota(jnp.int32, pad)
    src2d = jnp.concatenate([src, iota % N]).reshape(NW * CPT, CHUNK)
    dst2d = jnp.concatenate([dst, N + iota % (AGG_PAD - N)]).reshape(NW * CPT, CHUNK)
    zeros2d = jnp.zeros((AGG_PAD, D), jnp.float32)
    zeros1d = jnp.zeros((DEG_PAD,), jnp.float32)
    ones128 = jnp.ones((CHUNK,), jnp.float32)

    degp = _deg_sc(dst2d, zeros1d, ones128)
    d0 = degp[0, :N].reshape(N, 1)
    d1 = degp[1, :N].reshape(N, 1)

    h1, g1, dv = _dense1(x, W1, d0, d1)
    agg1 = _agg_sc(g1, src2d, dst2d, zeros2d)
    h2, g2 = _dense_mid(agg1[0, :N], agg1[1, :N], h1, dv,
                        b1.reshape(1, D), W2)
    agg2 = _agg_sc(g2, src2d, dst2d, zeros2d)
    return _dense_out(agg2[0, :N], agg2[1, :N], h2, dv,
                      b2.reshape(1, D), Wout, bout.reshape(1, 1))


# speedup vs baseline: 21.4629x; 1.0131x over previous
import functools

import jax
import jax.numpy as jnp
from jax import lax
from jax.experimental import pallas as pl
from jax.experimental.pallas import tpu as pltpu
from jax.experimental.pallas import tpu_sc as plsc

N = 10000
E = 320000
D = 128

NC = 2
NS = 16
NW = NC * NS

CHUNK = 128
CPT = 80
EPT = CPT * CHUNK
E_PAD = NW * EPT
NB = 4

AGG_PAD = 10240
AGG_TILE = AGG_PAD // NS
DEG_PAD = 10240
DEG_TILE = DEG_PAD // NS

R = 1000
G = N // R


def _mesh():
    return plsc.VectorSubcoreMesh(core_axis_name="c", subcore_axis_name="s")


@functools.partial(
    pl.kernel,
    out_type=jax.ShapeDtypeStruct((NC, DEG_PAD, 1), jnp.float32),
    mesh=_mesh(),
    scratch_types=[
        pltpu.VMEM((CPT, CHUNK), jnp.int32),
        pltpu.VMEM((CHUNK, 1), jnp.float32),
        pltpu.VMEM_SHARED((DEG_PAD, 1), jnp.float32),
    ],
)
def _deg_sc(dst_hbm, z_hbm, ones_hbm, out_hbm, dst_v, ones_v, acc_sh):
    cid = lax.axis_index("c")
    sid = lax.axis_index("s")
    wid = cid * NS + sid
    pltpu.sync_copy(dst_hbm.at[pl.ds(wid * CPT, CPT)], dst_v)
    pltpu.sync_copy(ones_hbm, ones_v)
    pltpu.sync_copy(z_hbm.at[pl.ds(sid * DEG_TILE, DEG_TILE)],
                    acc_sh.at[pl.ds(sid * DEG_TILE, DEG_TILE)])
    plsc.subcore_barrier()

    def body(j, carry):
        pltpu.sync_copy(ones_v, acc_sh.at[dst_v.at[j]], add=True)
        return carry

    lax.fori_loop(0, CPT, body, 0)
    plsc.subcore_barrier()
    pltpu.sync_copy(acc_sh.at[pl.ds(sid * DEG_TILE, DEG_TILE)],
                    out_hbm.at[cid].at[pl.ds(sid * DEG_TILE, DEG_TILE)])


@functools.partial(
    pl.kernel,
    out_type=jax.ShapeDtypeStruct((NC, AGG_PAD, D), jnp.float32),
    mesh=_mesh(),
    scratch_types=[
        pltpu.VMEM((CPT, CHUNK), jnp.int32),
        pltpu.VMEM((CPT, CHUNK), jnp.int32),
        pltpu.VMEM((CHUNK, D), jnp.float32),
        pltpu.SemaphoreType.DMA,
        pltpu.VMEM_SHARED((AGG_PAD, D), jnp.float32),
    ],
)
def _agg_sc(g_hbm, src_hbm, dst_hbm, z_hbm, out_hbm,
            src_v, dst_v, rows, gsem, acc_sh):
    cid = lax.axis_index("c")
    sid = lax.axis_index("s")
    wid = cid * NS + sid
    pltpu.sync_copy(src_hbm.at[pl.ds(wid * CPT, CPT)], src_v)
    pltpu.sync_copy(dst_hbm.at[pl.ds(wid * CPT, CPT)], dst_v)
    pltpu.sync_copy(z_hbm.at[pl.ds(sid * AGG_TILE, AGG_TILE)],
                    acc_sh.at[pl.ds(sid * AGG_TILE, AGG_TILE)])
    plsc.subcore_barrier()

    def body(j, carry):
        pltpu.async_copy(g_hbm.at[src_v.at[j]], rows, gsem).wait()
        pltpu.sync_copy(rows, acc_sh.at[dst_v.at[j]], add=True)
        return carry

    lax.fori_loop(0, CPT, body, 0)
    plsc.subcore_barrier()
    pltpu.sync_copy(acc_sh.at[pl.ds(sid * AGG_TILE, AGG_TILE)],
                    out_hbm.at[cid].at[pl.ds(sid * AGG_TILE, AGG_TILE)])


def _dense1(x, W1, degp):
    def body(x_ref, w_ref, d0_ref, d1_ref, h_ref, g_ref, dv_ref):
        dv = lax.rsqrt(d0_ref[0] + d1_ref[0] + 1.0)
        h = jnp.dot(x_ref[...], w_ref[...], preferred_element_type=jnp.float32)
        h_ref[...] = h
        g_ref[...] = h * dv
        dv_ref[...] = dv

    return pl.pallas_call(
        body,
        grid=(G,),
        in_specs=[
            pl.BlockSpec((R, D), lambda i: (i, 0)),
            pl.BlockSpec((D, D), lambda i: (0, 0)),
            pl.BlockSpec((1, R, 1), lambda i: (0, i, 0)),
            pl.BlockSpec((1, R, 1), lambda i: (1, i, 0)),
        ],
        out_specs=[
            pl.BlockSpec((R, D), lambda i: (i, 0)),
            pl.BlockSpec((R, D), lambda i: (i, 0)),
            pl.BlockSpec((R, 1), lambda i: (i, 0)),
        ],
        out_shape=[
            jax.ShapeDtypeStruct((N, D), jnp.float32),
            jax.ShapeDtypeStruct((N, D), jnp.float32),
            jax.ShapeDtypeStruct((N, 1), jnp.float32),
        ],
    )(x, W1, degp, degp)


def _dense_mid(agg, h1, dv, b1, W2):
    def body(p0_ref, p1_ref, h_ref, dv_ref, b_ref, w_ref, h2_ref, g2_ref):
        dv = dv_ref[...]
        z = dv * (p0_ref[0] + p1_ref[0]) + dv * dv * h_ref[...] + b_ref[...]
        z = jnp.maximum(z, 0.0)
        h2 = jnp.dot(z, w_ref[...], preferred_element_type=jnp.float32)
        h2_ref[...] = h2
        g2_ref[...] = h2 * dv

    return pl.pallas_call(
        body,
        grid=(G,),
        in_specs=[
            pl.BlockSpec((1, R, D), lambda i: (0, i, 0)),
            pl.BlockSpec((1, R, D), lambda i: (1, i, 0)),
            pl.BlockSpec((R, D), lambda i: (i, 0)),
            pl.BlockSpec((R, 1), lambda i: (i, 0)),
            pl.BlockSpec((1, D), lambda i: (0, 0)),
            pl.BlockSpec((D, D), lambda i: (0, 0)),
        ],
        out_specs=[
            pl.BlockSpec((R, D), lambda i: (i, 0)),
            pl.BlockSpec((R, D), lambda i: (i, 0)),
        ],
        out_shape=[
            jax.ShapeDtypeStruct((N, D), jnp.float32),
            jax.ShapeDtypeStruct((N, D), jnp.float32),
        ],
    )(agg, agg, h1, dv, b1, W2)


def _dense_out(agg, h2, dv, b2, Wout, bout):
    def body(q0_ref, q1_ref, h_ref, dv_ref, b_ref, w_ref, bo_ref, o_ref):
        dv = dv_ref[...]
        z = dv * (q0_ref[0] + q1_ref[0]) + dv * dv * h_ref[...] + b_ref[...]
        z = jnp.maximum(z, 0.0)
        o_ref[...] = jnp.dot(z, w_ref[...],
                             preferred_element_type=jnp.float32) + bo_ref[...]

    return pl.pallas_call(
        body,
        grid=(G,),
        in_specs=[
            pl.BlockSpec((1, R, D), lambda i: (0, i, 0)),
            pl.BlockSpec((1, R, D), lambda i: (1, i, 0)),
            pl.BlockSpec((R, D), lambda i: (i, 0)),
            pl.BlockSpec((R, 1), lambda i: (i, 0)),
            pl.BlockSpec((1, D), lambda i: (0, 0)),
            pl.BlockSpec((D, 1), lambda i: (0, 0)),
            pl.BlockSpec((1, 1), lambda i: (0, 0)),
        ],
        out_specs=pl.BlockSpec((R, 1), lambda i: (i, 0)),
        out_shape=jax.ShapeDtypeStruct((N, 1), jnp.float32),
    )(agg, agg, h2, dv, b2, Wout, bout)


def kernel(x, edge_index, W1, b1, W2, b2, Wout, bout):
    src = edge_index[0].astype(jnp.int32)
    dst = edge_index[1].astype(jnp.int32)
    pad = E_PAD - E
    iota = lax.iota(jnp.int32, pad)
    src2d = jnp.concatenate([src, iota % N]).reshape(NW * CPT, CHUNK)
    dst2d = jnp.concatenate([dst, N + iota % (AGG_PAD - N)]).reshape(NW * CPT, CHUNK)
    zeros2d = jnp.zeros((AGG_PAD, D), jnp.float32)
    zeros1d = jnp.zeros((DEG_PAD, 1), jnp.float32)
    ones128 = jnp.ones((CHUNK, 1), jnp.float32)

    degp = _deg_sc(dst2d, zeros1d, ones128)
    h1, g1, dv = _dense1(x, W1, degp)
    agg1 = _agg_sc(g1, src2d, dst2d, zeros2d)
    h2, g2 = _dense_mid(agg1, h1, dv, b1.reshape(1, D), W2)
    agg2 = _agg_sc(g2, src2d, dst2d, zeros2d)
    return _dense_out(agg2, h2, dv, b2.reshape(1, D), Wout, bout.reshape(1, 1))


# speedup vs baseline: 21.8277x; 1.0170x over previous
import functools

import jax
import jax.numpy as jnp
from jax import lax
from jax.experimental import pallas as pl
from jax.experimental.pallas import tpu as pltpu
from jax.experimental.pallas import tpu_sc as plsc

N = 10000
E = 320000
D = 128

NC = 2
NS = 16
NW = NC * NS

CHUNK = 128
CPT = 80
EPT = CPT * CHUNK
E_PAD = NW * EPT
NB = 4

AGG_PAD = 10240
AGG_TILE = AGG_PAD // NS
DEG_PAD = 10240
DEG_TILE = DEG_PAD // NS

R = 1000
G = N // R


def _mesh():
    return plsc.VectorSubcoreMesh(core_axis_name="c", subcore_axis_name="s")


@functools.partial(
    pl.kernel,
    out_type=jax.ShapeDtypeStruct((NC, DEG_PAD), jnp.float32),
    mesh=_mesh(),
    scratch_types=[
        pltpu.VMEM((CPT, CHUNK), jnp.int32),
        pltpu.VMEM((CHUNK,), jnp.float32),
        pltpu.VMEM_SHARED((DEG_PAD,), jnp.float32),
    ],
)
def _deg_sc(dst_hbm, z_hbm, ones_hbm, out_hbm, dst_v, ones_v, acc_sh):
    cid = lax.axis_index("c")
    sid = lax.axis_index("s")
    wid = cid * NS + sid
    pltpu.sync_copy(dst_hbm.at[pl.ds(wid * CPT, CPT)], dst_v)
    pltpu.sync_copy(ones_hbm, ones_v)
    pltpu.sync_copy(z_hbm.at[pl.ds(sid * DEG_TILE, DEG_TILE)],
                    acc_sh.at[pl.ds(sid * DEG_TILE, DEG_TILE)])
    plsc.subcore_barrier()

    def body(j, carry):
        pltpu.sync_copy(ones_v, acc_sh.at[dst_v.at[j]], add=True)
        return carry

    lax.fori_loop(0, CPT, body, 0)
    plsc.subcore_barrier()
    pltpu.sync_copy(acc_sh.at[pl.ds(sid * DEG_TILE, DEG_TILE)],
                    out_hbm.at[cid].at[pl.ds(sid * DEG_TILE, DEG_TILE)])


@functools.partial(
    pl.kernel,
    out_type=jax.ShapeDtypeStruct((NC, AGG_PAD, D), jnp.float32),
    mesh=_mesh(),
    scratch_types=[
        pltpu.VMEM((CPT, CHUNK), jnp.int32),
        pltpu.VMEM((CPT, CHUNK), jnp.int32),
        pltpu.VMEM((CHUNK, D), jnp.float32),
        pltpu.SemaphoreType.DMA,
        pltpu.VMEM_SHARED((AGG_PAD, D), jnp.float32),
    ],
)
def _agg_sc(g_hbm, src_hbm, dst_hbm, z_hbm, out_hbm,
            src_v, dst_v, rows, gsem, acc_sh):
    cid = lax.axis_index("c")
    sid = lax.axis_index("s")
    wid = cid * NS + sid
    pltpu.sync_copy(src_hbm.at[pl.ds(wid * CPT, CPT)], src_v)
    pltpu.sync_copy(dst_hbm.at[pl.ds(wid * CPT, CPT)], dst_v)
    pltpu.sync_copy(z_hbm.at[pl.ds(sid * AGG_TILE, AGG_TILE)],
                    acc_sh.at[pl.ds(sid * AGG_TILE, AGG_TILE)])
    plsc.subcore_barrier()

    def body(j, carry):
        pltpu.async_copy(g_hbm.at[src_v.at[j]], rows, gsem).wait()
        pltpu.sync_copy(rows, acc_sh.at[dst_v.at[j]], add=True)
        return carry

    lax.fori_loop(0, CPT, body, 0)
    plsc.subcore_barrier()
    pltpu.sync_copy(acc_sh.at[pl.ds(sid * AGG_TILE, AGG_TILE)],
                    out_hbm.at[cid].at[pl.ds(sid * AGG_TILE, AGG_TILE)])


def _dense1(x, W1, degp):
    def body(x_ref, w_ref, d0_ref, d1_ref, h_ref, g_ref, dv_ref):
        dv = lax.rsqrt(d0_ref[0] + d1_ref[0] + 1.0)
        h = jnp.dot(x_ref[...], w_ref[...], preferred_element_type=jnp.float32)
        h_ref[...] = h
        g_ref[...] = h * dv
        dv_ref[...] = dv

    return pl.pallas_call(
        body,
        grid=(G,),
        in_specs=[
            pl.BlockSpec((R, D), lambda i: (i, 0)),
            pl.BlockSpec((D, D), lambda i: (0, 0)),
            pl.BlockSpec((1, R, 1), lambda i: (0, i, 0)),
            pl.BlockSpec((1, R, 1), lambda i: (1, i, 0)),
        ],
        out_specs=[
            pl.BlockSpec((R, D), lambda i: (i, 0)),
            pl.BlockSpec((R, D), lambda i: (i, 0)),
            pl.BlockSpec((R, 1), lambda i: (i, 0)),
        ],
        out_shape=[
            jax.ShapeDtypeStruct((N, D), jnp.float32),
            jax.ShapeDtypeStruct((N, D), jnp.float32),
            jax.ShapeDtypeStruct((N, 1), jnp.float32),
        ],
    )(x, W1, degp, degp)


def _dense_mid(agg, h1, dv, b1, W2):
    def body(p0_ref, p1_ref, h_ref, dv_ref, b_ref, w_ref, h2_ref, g2_ref):
        dv = dv_ref[...]
        z = dv * (p0_ref[0] + p1_ref[0]) + dv * dv * h_ref[...] + b_ref[...]
        z = jnp.maximum(z, 0.0)
        h2 = jnp.dot(z, w_ref[...], preferred_element_type=jnp.float32)
        h2_ref[...] = h2
        g2_ref[...] = h2 * dv

    return pl.pallas_call(
        body,
        grid=(G,),
        in_specs=[
            pl.BlockSpec((1, R, D), lambda i: (0, i, 0)),
            pl.BlockSpec((1, R, D), lambda i: (1, i, 0)),
            pl.BlockSpec((R, D), lambda i: (i, 0)),
            pl.BlockSpec((R, 1), lambda i: (i, 0)),
            pl.BlockSpec((1, D), lambda i: (0, 0)),
            pl.BlockSpec((D, D), lambda i: (0, 0)),
        ],
        out_specs=[
            pl.BlockSpec((R, D), lambda i: (i, 0)),
            pl.BlockSpec((R, D), lambda i: (i, 0)),
        ],
        out_shape=[
            jax.ShapeDtypeStruct((N, D), jnp.float32),
            jax.ShapeDtypeStruct((N, D), jnp.float32),
        ],
    )(agg, agg, h1, dv, b1, W2)


def _dense_out(agg, h2, dv, b2, Wout, bout):
    def body(q0_ref, q1_ref, h_ref, dv_ref, b_ref, w_ref, bo_ref, o_ref):
        dv = dv_ref[...]
        z = dv * (q0_ref[0] + q1_ref[0]) + dv * dv * h_ref[...] + b_ref[...]
        z = jnp.maximum(z, 0.0)
        o_ref[...] = jnp.dot(z, w_ref[...],
                             preferred_element_type=jnp.float32) + bo_ref[...]

    return pl.pallas_call(
        body,
        grid=(G,),
        in_specs=[
            pl.BlockSpec((1, R, D), lambda i: (0, i, 0)),
            pl.BlockSpec((1, R, D), lambda i: (1, i, 0)),
            pl.BlockSpec((R, D), lambda i: (i, 0)),
            pl.BlockSpec((R, 1), lambda i: (i, 0)),
            pl.BlockSpec((1, D), lambda i: (0, 0)),
            pl.BlockSpec((D, 1), lambda i: (0, 0)),
            pl.BlockSpec((1, 1), lambda i: (0, 0)),
        ],
        out_specs=pl.BlockSpec((R, 1), lambda i: (i, 0)),
        out_shape=jax.ShapeDtypeStruct((N, 1), jnp.float32),
    )(agg, agg, h2, dv, b2, Wout, bout)


def kernel(x, edge_index, W1, b1, W2, b2, Wout, bout):
    src = edge_index[0].astype(jnp.int32)
    dst = edge_index[1].astype(jnp.int32)
    pad = E_PAD - E
    iota = lax.iota(jnp.int32, pad)
    src2d = jnp.concatenate([src, iota % N]).reshape(NW * CPT, CHUNK)
    dst2d = jnp.concatenate([dst, N + iota % (AGG_PAD - N)]).reshape(NW * CPT, CHUNK)
    zeros2d = jnp.zeros((AGG_PAD, D), jnp.float32)
    zeros1d = jnp.zeros((DEG_PAD,), jnp.float32)
    ones128 = jnp.ones((CHUNK,), jnp.float32)

    degp = _deg_sc(dst2d, zeros1d, ones128)
    h1, g1, dv = _dense1(x, W1, degp.reshape(NC, DEG_PAD, 1))
    agg1 = _agg_sc(g1, src2d, dst2d, zeros2d)
    h2, g2 = _dense_mid(agg1, h1, dv, b1.reshape(1, D), W2)
    agg2 = _agg_sc(g2, src2d, dst2d, zeros2d)
    return _dense_out(agg2, h2, dv, b2.reshape(1, D), Wout, bout.reshape(1, 1))


# speedup vs baseline: 21.8973x; 1.0032x over previous
import functools

import jax
import jax.numpy as jnp
from jax import lax
from jax.experimental import pallas as pl
from jax.experimental.pallas import tpu as pltpu
from jax.experimental.pallas import tpu_sc as plsc

N = 10000
E = 320000
D = 128

NC = 2
NS = 16
NW = NC * NS

CHUNK = 128
CPT = 80
EPT = CPT * CHUNK
E_PAD = NW * EPT
NB = 4

AGG_PAD = 10240
AGG_TILE = AGG_PAD // NS
DEG_PAD = 10240
DEG_TILE = DEG_PAD // NS

R = 1000
G = N // R


def _mesh():
    return plsc.VectorSubcoreMesh(core_axis_name="c", subcore_axis_name="s")


@functools.partial(
    pl.kernel,
    out_type=jax.ShapeDtypeStruct((NC, DEG_PAD), jnp.float32),
    mesh=_mesh(),
    scratch_types=[
        pltpu.VMEM((CPT, CHUNK), jnp.int32),
        pltpu.VMEM((CHUNK,), jnp.float32),
        pltpu.VMEM_SHARED((DEG_PAD,), jnp.float32),
    ],
)
def _deg_sc(dst_hbm, z_hbm, ones_hbm, out_hbm, dst_v, ones_v, acc_sh):
    cid = lax.axis_index("c")
    sid = lax.axis_index("s")
    wid = cid * NS + sid
    pltpu.sync_copy(dst_hbm.at[pl.ds(wid * CPT, CPT)], dst_v)
    pltpu.sync_copy(ones_hbm, ones_v)
    pltpu.sync_copy(z_hbm.at[pl.ds(sid * DEG_TILE, DEG_TILE)],
                    acc_sh.at[pl.ds(sid * DEG_TILE, DEG_TILE)])
    plsc.subcore_barrier()

    def body(j, carry):
        pltpu.sync_copy(ones_v, acc_sh.at[dst_v.at[j]], add=True)
        return carry

    lax.fori_loop(0, CPT, body, 0)
    plsc.subcore_barrier()
    pltpu.sync_copy(acc_sh.at[pl.ds(sid * DEG_TILE, DEG_TILE)],
                    out_hbm.at[cid].at[pl.ds(sid * DEG_TILE, DEG_TILE)])


@functools.partial(
    pl.kernel,
    out_type=jax.ShapeDtypeStruct((NC, AGG_PAD, D), jnp.float32),
    mesh=_mesh(),
    scratch_types=[
        pltpu.VMEM((CPT, CHUNK), jnp.int32),
        pltpu.VMEM((CPT, CHUNK), jnp.int32),
        pltpu.VMEM((CHUNK, D), jnp.float32),
        pltpu.SemaphoreType.DMA,
        pltpu.VMEM_SHARED((AGG_PAD, D), jnp.float32),
    ],
)
def _agg_sc(g_hbm, src_hbm, dst_hbm, z_hbm, out_hbm,
            src_v, dst_v, rows, gsem, acc_sh):
    cid = lax.axis_index("c")
    sid = lax.axis_index("s")
    wid = cid * NS + sid
    pltpu.sync_copy(src_hbm.at[pl.ds(wid * CPT, CPT)], src_v)
    pltpu.sync_copy(dst_hbm.at[pl.ds(wid * CPT, CPT)], dst_v)
    pltpu.sync_copy(z_hbm.at[pl.ds(sid * AGG_TILE, AGG_TILE)],
                    acc_sh.at[pl.ds(sid * AGG_TILE, AGG_TILE)])
    plsc.subcore_barrier()

    def body(j, carry):
        pltpu.async_copy(g_hbm.at[src_v.at[j]], rows, gsem).wait()
        pltpu.sync_copy(rows, acc_sh.at[dst_v.at[j]], add=True)
        return carry

    lax.fori_loop(0, CPT, body, 0)
    plsc.subcore_barrier()
    pltpu.sync_copy(acc_sh.at[pl.ds(sid * AGG_TILE, AGG_TILE)],
                    out_hbm.at[cid].at[pl.ds(sid * AGG_TILE, AGG_TILE)])


def _dense1(x, W1, degp):
    def body(x_ref, w_ref, d0_ref, d1_ref, h_ref, g_ref, dv_ref):
        dv = lax.rsqrt(d0_ref[0] + d1_ref[0] + 1.0)
        h = jnp.dot(x_ref[...], w_ref[...], preferred_element_type=jnp.float32)
        h_ref[...] = h
        g_ref[...] = h * dv
        dv_ref[...] = jnp.broadcast_to(dv, (R, D))

    return pl.pallas_call(
        body,
        grid=(G,),
        in_specs=[
            pl.BlockSpec((R, D), lambda i: (i, 0)),
            pl.BlockSpec((D, D), lambda i: (0, 0)),
            pl.BlockSpec((1, R, 1), lambda i: (0, i, 0)),
            pl.BlockSpec((1, R, 1), lambda i: (1, i, 0)),
        ],
        out_specs=[
            pl.BlockSpec((R, D), lambda i: (i, 0)),
            pl.BlockSpec((R, D), lambda i: (i, 0)),
            pl.BlockSpec((R, D), lambda i: (i, 0)),
        ],
        out_shape=[
            jax.ShapeDtypeStruct((N, D), jnp.float32),
            jax.ShapeDtypeStruct((N, D), jnp.float32),
            jax.ShapeDtypeStruct((N, D), jnp.float32),
        ],
    )(x, W1, degp, degp)


def _dense_mid(agg, h1, dv, b1, W2):
    def body(p0_ref, p1_ref, h_ref, dv_ref, b_ref, w_ref, h2_ref, g2_ref):
        dv = dv_ref[...]
        z = dv * (p0_ref[0] + p1_ref[0]) + dv * dv * h_ref[...] + b_ref[...]
        z = jnp.maximum(z, 0.0)
        h2 = jnp.dot(z, w_ref[...], preferred_element_type=jnp.float32)
        h2_ref[...] = h2
        g2_ref[...] = h2 * dv

    return pl.pallas_call(
        body,
        grid=(G,),
        in_specs=[
            pl.BlockSpec((1, R, D), lambda i: (0, i, 0)),
            pl.BlockSpec((1, R, D), lambda i: (1, i, 0)),
            pl.BlockSpec((R, D), lambda i: (i, 0)),
            pl.BlockSpec((R, D), lambda i: (i, 0)),
            pl.BlockSpec((1, D), lambda i: (0, 0)),
            pl.BlockSpec((D, D), lambda i: (0, 0)),
        ],
        out_specs=[
            pl.BlockSpec((R, D), lambda i: (i, 0)),
            pl.BlockSpec((R, D), lambda i: (i, 0)),
        ],
        out_shape=[
            jax.ShapeDtypeStruct((N, D), jnp.float32),
            jax.ShapeDtypeStruct((N, D), jnp.float32),
        ],
    )(agg, agg, h1, dv, b1, W2)


def _dense_out(agg, h2, dv, b2, Wout, bout):
    def body(q0_ref, q1_ref, h_ref, dv_ref, b_ref, w_ref, bo_ref, o_ref):
        dv = dv_ref[...]
        z = dv * (q0_ref[0] + q1_ref[0]) + dv * dv * h_ref[...] + b_ref[...]
        z = jnp.maximum(z, 0.0)
        o_ref[...] = jnp.dot(z, w_ref[...],
                             preferred_element_type=jnp.float32) + bo_ref[...]

    return pl.pallas_call(
        body,
        grid=(G,),
        in_specs=[
            pl.BlockSpec((1, R, D), lambda i: (0, i, 0)),
            pl.BlockSpec((1, R, D), lambda i: (1, i, 0)),
            pl.BlockSpec((R, D), lambda i: (i, 0)),
            pl.BlockSpec((R, D), lambda i: (i, 0)),
            pl.BlockSpec((1, D), lambda i: (0, 0)),
            pl.BlockSpec((D, 1), lambda i: (0, 0)),
            pl.BlockSpec((1, 1), lambda i: (0, 0)),
        ],
        out_specs=pl.BlockSpec((R, 1), lambda i: (i, 0)),
        out_shape=jax.ShapeDtypeStruct((N, 1), jnp.float32),
    )(agg, agg, h2, dv, b2, Wout, bout)


def kernel(x, edge_index, W1, b1, W2, b2, Wout, bout):
    src = edge_index[0].astype(jnp.int32)
    dst = edge_index[1].astype(jnp.int32)
    pad = E_PAD - E
    iota = lax.iota(jnp.int32, pad)
    src2d = jnp.concatenate([src, iota % N]).reshape(NW * CPT, CHUNK)
    dst2d = jnp.concatenate([dst, N + iota % (AGG_PAD - N)]).reshape(NW * CPT, CHUNK)
    zeros2d = jnp.zeros((AGG_PAD, D), jnp.float32)
    zeros1d = jnp.zeros((DEG_PAD,), jnp.float32)
    ones128 = jnp.ones((CHUNK,), jnp.float32)

    degp = _deg_sc(dst2d, zeros1d, ones128)
    h1, g1, dv = _dense1(x, W1, degp.reshape(NC, DEG_PAD, 1))
    agg1 = _agg_sc(g1, src2d, dst2d, zeros2d)
    h2, g2 = _dense_mid(agg1, h1, dv, b1.reshape(1, D), W2)
    agg2 = _agg_sc(g2, src2d, dst2d, zeros2d)
    return _dense_out(agg2, h2, dv, b2.reshape(1, D), Wout, bout.reshape(1, 1))


# speedup vs baseline: 22.2632x; 1.0167x over previous
import functools

import jax
import jax.numpy as jnp
from jax import lax
from jax.experimental import pallas as pl
from jax.experimental.pallas import tpu as pltpu
from jax.experimental.pallas import tpu_sc as plsc

N = 10000
E = 320000
D = 128

NC = 2
NS = 16
NW = NC * NS

CHUNK = 128
CPT = 80
EPT = CPT * CHUNK
E_PAD = NW * EPT
NB = 4

AGG_PAD = 10240
AGG_TILE = AGG_PAD // NS
DEG_PAD = 10240
DEG_TILE = DEG_PAD // NS

R = 2000
G = N // R


def _mesh():
    return plsc.VectorSubcoreMesh(core_axis_name="c", subcore_axis_name="s")


@functools.partial(
    pl.kernel,
    out_type=jax.ShapeDtypeStruct((NC, DEG_PAD), jnp.float32),
    mesh=_mesh(),
    scratch_types=[
        pltpu.VMEM((CPT, CHUNK), jnp.int32),
        pltpu.VMEM((CHUNK,), jnp.float32),
        pltpu.VMEM_SHARED((DEG_PAD,), jnp.float32),
    ],
)
def _deg_sc(dst_hbm, z_hbm, ones_hbm, out_hbm, dst_v, ones_v, acc_sh):
    cid = lax.axis_index("c")
    sid = lax.axis_index("s")
    wid = cid * NS + sid
    pltpu.sync_copy(dst_hbm.at[pl.ds(wid * CPT, CPT)], dst_v)
    pltpu.sync_copy(ones_hbm, ones_v)
    pltpu.sync_copy(z_hbm.at[pl.ds(sid * DEG_TILE, DEG_TILE)],
                    acc_sh.at[pl.ds(sid * DEG_TILE, DEG_TILE)])
    plsc.subcore_barrier()

    def body(j, carry):
        pltpu.sync_copy(ones_v, acc_sh.at[dst_v.at[j]], add=True)
        return carry

    lax.fori_loop(0, CPT, body, 0)
    plsc.subcore_barrier()
    pltpu.sync_copy(acc_sh.at[pl.ds(sid * DEG_TILE, DEG_TILE)],
                    out_hbm.at[cid].at[pl.ds(sid * DEG_TILE, DEG_TILE)])


@functools.partial(
    pl.kernel,
    out_type=jax.ShapeDtypeStruct((NC, AGG_PAD, D), jnp.float32),
    mesh=_mesh(),
    scratch_types=[
        pltpu.VMEM((CPT, CHUNK), jnp.int32),
        pltpu.VMEM((CPT, CHUNK), jnp.int32),
        pltpu.VMEM((CHUNK, D), jnp.float32),
        pltpu.SemaphoreType.DMA,
        pltpu.VMEM_SHARED((AGG_PAD, D), jnp.float32),
    ],
)
def _agg_sc(g_hbm, src_hbm, dst_hbm, z_hbm, out_hbm,
            src_v, dst_v, rows, gsem, acc_sh):
    cid = lax.axis_index("c")
    sid = lax.axis_index("s")
    wid = cid * NS + sid
    pltpu.sync_copy(src_hbm.at[pl.ds(wid * CPT, CPT)], src_v)
    pltpu.sync_copy(dst_hbm.at[pl.ds(wid * CPT, CPT)], dst_v)
    pltpu.sync_copy(z_hbm.at[pl.ds(sid * AGG_TILE, AGG_TILE)],
                    acc_sh.at[pl.ds(sid * AGG_TILE, AGG_TILE)])
    plsc.subcore_barrier()

    def body(step, carry):
        for k in range(2):
            j = step * 2 + k
            pltpu.async_copy(g_hbm.at[src_v.at[j]], rows, gsem).wait()
            pltpu.sync_copy(rows, acc_sh.at[dst_v.at[j]], add=True)
        return carry

    lax.fori_loop(0, CPT // 2, body, 0)
    plsc.subcore_barrier()
    pltpu.sync_copy(acc_sh.at[pl.ds(sid * AGG_TILE, AGG_TILE)],
                    out_hbm.at[cid].at[pl.ds(sid * AGG_TILE, AGG_TILE)])


def _dense1(x, W1, degp):
    def body(x_ref, w_ref, d0_ref, d1_ref, h_ref, g_ref, dv_ref):
        dv = lax.rsqrt(d0_ref[0] + d1_ref[0] + 1.0)
        h = jnp.dot(x_ref[...], w_ref[...], preferred_element_type=jnp.float32)
        h_ref[...] = h
        g_ref[...] = h * dv
        dv_ref[...] = jnp.broadcast_to(dv, (R, D))

    return pl.pallas_call(
        body,
        grid=(G,),
        in_specs=[
            pl.BlockSpec((R, D), lambda i: (i, 0)),
            pl.BlockSpec((D, D), lambda i: (0, 0)),
            pl.BlockSpec((1, R, 1), lambda i: (0, i, 0)),
            pl.BlockSpec((1, R, 1), lambda i: (1, i, 0)),
        ],
        out_specs=[
            pl.BlockSpec((R, D), lambda i: (i, 0)),
            pl.BlockSpec((R, D), lambda i: (i, 0)),
            pl.BlockSpec((R, D), lambda i: (i, 0)),
        ],
        out_shape=[
            jax.ShapeDtypeStruct((N, D), jnp.float32),
            jax.ShapeDtypeStruct((N, D), jnp.float32),
            jax.ShapeDtypeStruct((N, D), jnp.float32),
        ],
    )(x, W1, degp, degp)


def _dense_mid(agg, h1, dv, b1, W2):
    def body(p0_ref, p1_ref, h_ref, dv_ref, b_ref, w_ref, h2_ref, g2_ref):
        dv = dv_ref[...]
        z = dv * (p0_ref[0] + p1_ref[0]) + dv * dv * h_ref[...] + b_ref[...]
        z = jnp.maximum(z, 0.0)
        h2 = jnp.dot(z, w_ref[...], preferred_element_type=jnp.float32)
        h2_ref[...] = h2
        g2_ref[...] = h2 * dv

    return pl.pallas_call(
        body,
        grid=(G,),
        in_specs=[
            pl.BlockSpec((1, R, D), lambda i: (0, i, 0)),
            pl.BlockSpec((1, R, D), lambda i: (1, i, 0)),
            pl.BlockSpec((R, D), lambda i: (i, 0)),
            pl.BlockSpec((R, D), lambda i: (i, 0)),
            pl.BlockSpec((1, D), lambda i: (0, 0)),
            pl.BlockSpec((D, D), lambda i: (0, 0)),
        ],
        out_specs=[
            pl.BlockSpec((R, D), lambda i: (i, 0)),
            pl.BlockSpec((R, D), lambda i: (i, 0)),
        ],
        out_shape=[
            jax.ShapeDtypeStruct((N, D), jnp.float32),
            jax.ShapeDtypeStruct((N, D), jnp.float32),
        ],
    )(agg, agg, h1, dv, b1, W2)


def _dense_out(agg, h2, dv, b2, Wout, bout):
    def body(q0_ref, q1_ref, h_ref, dv_ref, b_ref, w_ref, bo_ref, o_ref):
        dv = dv_ref[...]
        z = dv * (q0_ref[0] + q1_ref[0]) + dv * dv * h_ref[...] + b_ref[...]
        z = jnp.maximum(z, 0.0)
        o_ref[...] = jnp.dot(z, w_ref[...],
                             preferred_element_type=jnp.float32) + bo_ref[...]

    return pl.pallas_call(
        body,
        grid=(G,),
        in_specs=[
            pl.BlockSpec((1, R, D), lambda i: (0, i, 0)),
            pl.BlockSpec((1, R, D), lambda i: (1, i, 0)),
            pl.BlockSpec((R, D), lambda i: (i, 0)),
            pl.BlockSpec((R, D), lambda i: (i, 0)),
            pl.BlockSpec((1, D), lambda i: (0, 0)),
            pl.BlockSpec((D, 1), lambda i: (0, 0)),
            pl.BlockSpec((1, 1), lambda i: (0, 0)),
        ],
        out_specs=pl.BlockSpec((R, 1), lambda i: (i, 0)),
        out_shape=jax.ShapeDtypeStruct((N, 1), jnp.float32),
    )(agg, agg, h2, dv, b2, Wout, bout)


def kernel(x, edge_index, W1, b1, W2, b2, Wout, bout):
    src = edge_index[0].astype(jnp.int32)
    dst = edge_index[1].astype(jnp.int32)
    pad = E_PAD - E
    iota = lax.iota(jnp.int32, pad)
    src2d = jnp.concatenate([src, iota % N]).reshape(NW * CPT, CHUNK)
    dst2d = jnp.concatenate([dst, N + iota % (AGG_PAD - N)]).reshape(NW * CPT, CHUNK)
    zeros2d = jnp.zeros((AGG_PAD, D), jnp.float32)
    zeros1d = jnp.zeros((DEG_PAD,), jnp.float32)
    ones128 = jnp.ones((CHUNK,), jnp.float32)

    degp = _deg_sc(dst2d, zeros1d, ones128)
    h1, g1, dv = _dense1(x, W1, degp.reshape(NC, DEG_PAD, 1))
    agg1 = _agg_sc(g1, src2d, dst2d, zeros2d)
    h2, g2 = _dense_mid(agg1, h1, dv, b1.reshape(1, D), W2)
    agg2 = _agg_sc(g2, src2d, dst2d, zeros2d)
    return _dense_out(agg2, h2, dv, b2.reshape(1, D), Wout, bout.reshape(1, 1))
